# Initial kernel scaffold; baseline (speedup 1.0000x reference)
import jax
import jax.numpy as jnp
from jax.experimental import pallas as pl


def kernel(i, x, y):
    raise NotImplementedError("write your pallas kernel here")


# speedup vs baseline: 4.6779x; 4.6779x over previous
import functools

import jax
import jax.numpy as jnp
from jax.experimental import pallas as pl
from jax.experimental.pallas import tpu as pltpu

_TM = 1024
_TN = 1024


def _addmm_kernel(i_ref, x_ref, y_ref, o_ref, *, beta, alpha):
    acc = jnp.dot(x_ref[...], y_ref[...], preferred_element_type=jnp.float32)
    o_ref[...] = beta * i_ref[...] + alpha * acc


def kernel(i, x, y):
    beta, alpha = 1.0, 2.0
    M, K = x.shape
    _, N = y.shape

    xb = x.astype(jnp.bfloat16)
    yb = y.astype(jnp.bfloat16)
    i2 = i.reshape(1, N)

    kfn = functools.partial(_addmm_kernel, beta=beta, alpha=alpha)
    return pl.pallas_call(
        kfn,
        out_shape=jax.ShapeDtypeStruct((M, N), jnp.float32),
        grid=(M // _TM, N // _TN),
        in_specs=[
            pl.BlockSpec((1, _TN), lambda m, n: (0, n)),
            pl.BlockSpec((_TM, K), lambda m, n: (m, 0)),
            pl.BlockSpec((K, _TN), lambda m, n: (0, n)),
        ],
        out_specs=pl.BlockSpec((_TM, _TN), lambda m, n: (m, n)),
        compiler_params=pltpu.CompilerParams(
            dimension_semantics=("parallel", "parallel")
        ),
    )(i2, xb, yb)


# speedup vs baseline: 5.7979x; 1.2394x over previous
import functools

import jax
import jax.numpy as jnp
from jax.experimental import pallas as pl
from jax.experimental.pallas import tpu as pltpu

_TM = 512
_TN = 1024


def _addmm_kernel(i_ref, x_ref, y_ref, o_ref, *, beta, alpha):
    acc = jnp.dot(x_ref[...], y_ref[...], preferred_element_type=jnp.float32)
    o_ref[...] = beta * i_ref[...] + alpha * acc


def kernel(i, x, y):
    beta, alpha = 1.0, 2.0
    M, K = x.shape
    _, N = y.shape
    i2 = i.reshape(1, N)

    kfn = functools.partial(_addmm_kernel, beta=beta, alpha=alpha)
    return pl.pallas_call(
        kfn,
        out_shape=jax.ShapeDtypeStruct((M, N), jnp.float32),
        grid=(N // _TN, M // _TM),
        in_specs=[
            pl.BlockSpec((1, _TN), lambda n, m: (0, n)),
            pl.BlockSpec((_TM, K), lambda n, m: (m, 0)),
            pl.BlockSpec((K, _TN), lambda n, m: (0, n)),
        ],
        out_specs=pl.BlockSpec((_TM, _TN), lambda n, m: (m, n)),
        compiler_params=pltpu.CompilerParams(
            dimension_semantics=("parallel", "parallel")
        ),
    )(i2, x, y)
